# baseline (speedup 1.0000x reference)
import jax
import jax.numpy as jnp
from jax.experimental import pallas as pl
from jax.experimental.pallas import tpu as pltpu

N = 4096
K = 4096
CH = 256
NCHUNKS = N // CH
NBUF = 2
NOBUF = 2


def _mm_kernel(x_hbm, b_hbm, w_hbm, o_hbm, xbuf, bbuf, wbuf, obuf,
               wsems, osems, xsem, bsem):
    def wcopy(c):
        return pltpu.make_async_copy(
            w_hbm.at[pl.ds(c * CH, CH)], wbuf.at[c % NBUF], wsems.at[c % NBUF])

    def ocopy(c):
        return pltpu.make_async_copy(
            obuf.at[c % NOBUF], o_hbm.at[:, pl.ds(c * CH, CH)],
            osems.at[c % NOBUF])

    pltpu.make_async_copy(x_hbm, xbuf, xsem).start()
    pltpu.make_async_copy(b_hbm, bbuf, bsem).start()
    for c in range(NBUF):
        wcopy(c).start()
    pltpu.make_async_copy(x_hbm, xbuf, xsem).wait()
    pltpu.make_async_copy(b_hbm, bbuf, bsem).wait()
    for c in range(NCHUNKS):
        wcopy(c).wait()
        if c >= NOBUF:
            ocopy(c - NOBUF).wait()
        acc = jax.lax.dot_general(
            xbuf[...], wbuf[c % NBUF],
            dimension_numbers=(((1,), (1,)), ((), ())),
            preferred_element_type=jnp.float32,
        )
        obuf[c % NOBUF] = acc + bbuf[:, c * CH:(c + 1) * CH]
        ocopy(c).start()
        if c + NBUF < NCHUNKS:
            wcopy(c + NBUF).start()
    for c in range(NCHUNKS - NOBUF, NCHUNKS):
        ocopy(c).wait()


@jax.jit
def kernel(x, weight, bias):
    m = x.shape[0]
    bias2d = bias.reshape(1, N)
    out = pl.pallas_call(
        _mm_kernel,
        in_specs=[
            pl.BlockSpec(memory_space=pltpu.MemorySpace.HBM),
            pl.BlockSpec(memory_space=pltpu.MemorySpace.HBM),
            pl.BlockSpec(memory_space=pltpu.MemorySpace.HBM),
        ],
        out_specs=pl.BlockSpec(memory_space=pltpu.MemorySpace.HBM),
        out_shape=jax.ShapeDtypeStruct((m, N), jnp.float32),
        scratch_shapes=[
            pltpu.VMEM((64, K), jnp.float32),
            pltpu.VMEM((1, N), jnp.float32),
            pltpu.VMEM((NBUF, CH, K), jnp.float32),
            pltpu.VMEM((NOBUF, 64, CH), jnp.float32),
            pltpu.SemaphoreType.DMA((NBUF,)),
            pltpu.SemaphoreType.DMA((NOBUF,)),
            pltpu.SemaphoreType.DMA,
            pltpu.SemaphoreType.DMA,
        ],
    )(x, bias2d, weight)
    return out


# speedup vs baseline: 1.1747x; 1.1747x over previous
import jax
import jax.numpy as jnp
from jax.experimental import pallas as pl
from jax.experimental.pallas import tpu as pltpu

N = 4096
K = 4096
CHUNKS = (256, 256, 512, 512, 512, 512, 512, 512, 256, 256)
OFFS = tuple(sum(CHUNKS[:i]) for i in range(len(CHUNKS)))
NCHUNKS = len(CHUNKS)
CHMAX = max(CHUNKS)
NBUF = 2
NOBUF = 2


def _mm_kernel(x_hbm, b_hbm, w_hbm, o_hbm, xbuf, bbuf, wbuf, obuf,
               wsems, osems, xsem, bsem):
    def wcopy(c):
        return pltpu.make_async_copy(
            w_hbm.at[pl.ds(OFFS[c], CHUNKS[c])],
            wbuf.at[c % NBUF, pl.ds(0, CHUNKS[c])],
            wsems.at[c % NBUF])

    def ocopy(c):
        return pltpu.make_async_copy(
            obuf.at[c % NOBUF, :, pl.ds(0, CHUNKS[c])],
            o_hbm.at[:, pl.ds(OFFS[c], CHUNKS[c])],
            osems.at[c % NOBUF])

    pltpu.make_async_copy(x_hbm, xbuf, xsem).start()
    pltpu.make_async_copy(b_hbm, bbuf, bsem).start()
    for c in range(NBUF):
        wcopy(c).start()
    pltpu.make_async_copy(x_hbm, xbuf, xsem).wait()
    pltpu.make_async_copy(b_hbm, bbuf, bsem).wait()
    for c in range(NCHUNKS):
        wcopy(c).wait()
        if c >= NOBUF:
            ocopy(c - NOBUF).wait()
        acc = jax.lax.dot_general(
            xbuf[...], wbuf[c % NBUF, :CHUNKS[c]],
            dimension_numbers=(((1,), (1,)), ((), ())),
            preferred_element_type=jnp.float32,
        )
        obuf[c % NOBUF, :, :CHUNKS[c]] = (
            acc + bbuf[:, OFFS[c]:OFFS[c] + CHUNKS[c]])
        ocopy(c).start()
        if c + NBUF < NCHUNKS:
            wcopy(c + NBUF).start()
    for c in range(NCHUNKS - NOBUF, NCHUNKS):
        ocopy(c).wait()


@jax.jit
def kernel(x, weight, bias):
    m = x.shape[0]
    bias2d = bias.reshape(1, N)
    out = pl.pallas_call(
        _mm_kernel,
        in_specs=[
            pl.BlockSpec(memory_space=pltpu.MemorySpace.HBM),
            pl.BlockSpec(memory_space=pltpu.MemorySpace.HBM),
            pl.BlockSpec(memory_space=pltpu.MemorySpace.HBM),
        ],
        out_specs=pl.BlockSpec(memory_space=pltpu.MemorySpace.HBM),
        out_shape=jax.ShapeDtypeStruct((m, N), jnp.float32),
        scratch_shapes=[
            pltpu.VMEM((64, K), jnp.float32),
            pltpu.VMEM((1, N), jnp.float32),
            pltpu.VMEM((NBUF, CHMAX, K), jnp.float32),
            pltpu.VMEM((NOBUF, 64, CHMAX), jnp.float32),
            pltpu.SemaphoreType.DMA((NBUF,)),
            pltpu.SemaphoreType.DMA((NOBUF,)),
            pltpu.SemaphoreType.DMA,
            pltpu.SemaphoreType.DMA,
        ],
    )(x, bias2d, weight)
    return out


# speedup vs baseline: 1.2633x; 1.0755x over previous
import jax
import jax.numpy as jnp
from jax.experimental import pallas as pl
from jax.experimental.pallas import tpu as pltpu

N = 4096
K = 4096
CHUNKS = (512, 512, 512, 512, 512, 512, 512, 512)
OFFS = tuple(sum(CHUNKS[:i]) for i in range(len(CHUNKS)))
NCHUNKS = len(CHUNKS)
CHMAX = max(CHUNKS)
NBUF = 2
NOBUF = 2


def _mm_kernel(x_hbm, b_hbm, w_hbm, o_hbm, xbuf, bbuf, wbuf, obuf,
               wsems, osems, xsem, bsem):
    def wcopy(c):
        return pltpu.make_async_copy(
            w_hbm.at[pl.ds(OFFS[c], CHUNKS[c])],
            wbuf.at[c % NBUF, pl.ds(0, CHUNKS[c])],
            wsems.at[c % NBUF])

    def ocopy(c):
        return pltpu.make_async_copy(
            obuf.at[c % NOBUF, :, pl.ds(0, CHUNKS[c])],
            o_hbm.at[:, pl.ds(OFFS[c], CHUNKS[c])],
            osems.at[c % NOBUF])

    pltpu.make_async_copy(x_hbm, xbuf, xsem).start()
    pltpu.make_async_copy(b_hbm, bbuf, bsem).start()
    for c in range(NBUF):
        wcopy(c).start()
    pltpu.make_async_copy(x_hbm, xbuf, xsem).wait()
    pltpu.make_async_copy(b_hbm, bbuf, bsem).wait()
    for c in range(NCHUNKS):
        wcopy(c).wait()
        if c >= NOBUF:
            ocopy(c - NOBUF).wait()
        acc = jax.lax.dot_general(
            xbuf[...], wbuf[c % NBUF, :CHUNKS[c]],
            dimension_numbers=(((1,), (1,)), ((), ())),
            preferred_element_type=jnp.float32,
        )
        obuf[c % NOBUF, :, :CHUNKS[c]] = (
            acc + bbuf[:, OFFS[c]:OFFS[c] + CHUNKS[c]])
        ocopy(c).start()
        if c + NBUF < NCHUNKS:
            wcopy(c + NBUF).start()
    for c in range(NCHUNKS - NOBUF, NCHUNKS):
        ocopy(c).wait()


@jax.jit
def kernel(x, weight, bias):
    m = x.shape[0]
    bias2d = bias.reshape(1, N)
    out = pl.pallas_call(
        _mm_kernel,
        in_specs=[
            pl.BlockSpec(memory_space=pltpu.MemorySpace.HBM),
            pl.BlockSpec(memory_space=pltpu.MemorySpace.HBM),
            pl.BlockSpec(memory_space=pltpu.MemorySpace.HBM),
        ],
        out_specs=pl.BlockSpec(memory_space=pltpu.MemorySpace.HBM),
        out_shape=jax.ShapeDtypeStruct((m, N), jnp.float32),
        scratch_shapes=[
            pltpu.VMEM((64, K), jnp.float32),
            pltpu.VMEM((1, N), jnp.float32),
            pltpu.VMEM((NBUF, CHMAX, K), jnp.float32),
            pltpu.VMEM((NOBUF, 64, CHMAX), jnp.float32),
            pltpu.SemaphoreType.DMA((NBUF,)),
            pltpu.SemaphoreType.DMA((NOBUF,)),
            pltpu.SemaphoreType.DMA,
            pltpu.SemaphoreType.DMA,
        ],
    )(x, bias2d, weight)
    return out
